# baseline (speedup 1.0000x reference)
import jax
import jax.numpy as jnp
from jax.experimental import pallas as pl
from jax.experimental.pallas import tpu as pltpu

_B, _H, _W = 32, 56, 56
_NP, _E, _O = 4, 64, 256
_HW = _H * _W


def _period_kernel(labels_ref, emb_ref, w_ref, b_ref, out_ref, tiles_ref,
                   sems):
    feats = jax.lax.dot_general(
        emb_ref[:], w_ref[:], (((1,), (1,)), ((), ())),
        preferred_element_type=jnp.float32) + b_ref[:]
    q4 = _HW // 4
    for p in range(_NP):
        if p == 0:
            for q in range(4):
                tiles_ref[0, pl.ds(q * q4, q4)] = jnp.broadcast_to(
                    feats[0, None, :], (q4, _O))
                for b in range(_B):
                    @pl.when(labels_ref[b] == 0)
                    def _start_chunk():
                        pltpu.make_async_copy(
                            tiles_ref.at[0, pl.ds(q * q4, q4)],
                            out_ref.at[b, pl.ds(q * q4, q4)],
                            sems.at[b]).start()
        else:
            tiles_ref[p] = jnp.broadcast_to(feats[p, None, :], (_HW, _O))
            for b in range(_B):
                @pl.when(labels_ref[b] == p)
                def _start():
                    pltpu.make_async_copy(
                        tiles_ref.at[p], out_ref.at[b], sems.at[b]).start()
    for b in range(_B):
        for q in range(4):
            pltpu.make_async_copy(
                tiles_ref.at[0, pl.ds(q * q4, q4)],
                out_ref.at[b, pl.ds(q * q4, q4)],
                sems.at[b]).wait()


def kernel(period_labels, spatial_size, emb_table, fc_w, fc_b):
    fcb2d = fc_b.reshape(1, _O)
    grid_spec = pltpu.PrefetchScalarGridSpec(
        num_scalar_prefetch=1,
        grid=(1,),
        in_specs=[
            pl.BlockSpec((_NP, _E), lambda i, s: (0, 0)),
            pl.BlockSpec((_O, _E), lambda i, s: (0, 0)),
            pl.BlockSpec((1, _O), lambda i, s: (0, 0)),
        ],
        out_specs=pl.BlockSpec(memory_space=pl.ANY),
        scratch_shapes=[
            pltpu.VMEM((_NP, _HW, _O), jnp.float32),
            pltpu.SemaphoreType.DMA((_B,)),
        ],
    )
    out = pl.pallas_call(
        _period_kernel,
        grid_spec=grid_spec,
        out_shape=jax.ShapeDtypeStruct((_B, _HW, _O), jnp.float32),
    )(period_labels.astype(jnp.int32), emb_table, fc_w, fcb2d)
    out = out.reshape(_B, _H, _W, _O)
    return jnp.transpose(out, (0, 3, 1, 2))


# speedup vs baseline: 1.0052x; 1.0052x over previous
import jax
import jax.numpy as jnp
from jax.experimental import pallas as pl
from jax.experimental.pallas import tpu as pltpu

_B, _H, _W = 32, 56, 56
_NP, _E, _O = 4, 64, 256
_HW = _H * _W


def _period_kernel(labels_ref, emb_ref, w_ref, b_ref, out_ref, tiles_ref,
                   sems):
    feats = jax.lax.dot_general(
        emb_ref[:], w_ref[:], (((1,), (1,)), ((), ())),
        preferred_element_type=jnp.float32) + b_ref[:]
    q4 = _HW // 4
    for p in range(_NP):
        if p == 0:
            for q in range(4):
                tiles_ref[0, pl.ds(q * q4, q4)] = jnp.broadcast_to(
                    feats[0, None, :], (q4, _O))
                for b in range(_B):
                    @pl.when(labels_ref[b] == 0)
                    def _start_chunk():
                        pltpu.make_async_copy(
                            tiles_ref.at[0, pl.ds(q * q4, q4)],
                            out_ref.at[b, pl.ds(q * q4, q4)],
                            sems.at[b]).start()
        else:
            tiles_ref[p] = jnp.broadcast_to(feats[p, None, :], (_HW, _O))
            for b in range(_B):
                @pl.when(labels_ref[b] == p)
                def _start():
                    pltpu.make_async_copy(
                        tiles_ref.at[p], out_ref.at[b], sems.at[b]).start()
    for b in range(_B):
        pltpu.make_async_copy(
            tiles_ref.at[0], out_ref.at[b], sems.at[b]).wait()


def kernel(period_labels, spatial_size, emb_table, fc_w, fc_b):
    fcb2d = fc_b.reshape(1, _O)
    grid_spec = pltpu.PrefetchScalarGridSpec(
        num_scalar_prefetch=1,
        grid=(1,),
        in_specs=[
            pl.BlockSpec((_NP, _E), lambda i, s: (0, 0)),
            pl.BlockSpec((_O, _E), lambda i, s: (0, 0)),
            pl.BlockSpec((1, _O), lambda i, s: (0, 0)),
        ],
        out_specs=pl.BlockSpec(memory_space=pl.ANY),
        scratch_shapes=[
            pltpu.VMEM((_NP, _HW, _O), jnp.float32),
            pltpu.SemaphoreType.DMA((_B,)),
        ],
    )
    out = pl.pallas_call(
        _period_kernel,
        grid_spec=grid_spec,
        out_shape=jax.ShapeDtypeStruct((_B, _HW, _O), jnp.float32),
    )(period_labels.astype(jnp.int32), emb_table, fc_w, fcb2d)
    out = out.reshape(_B, _H, _W, _O)
    return jnp.transpose(out, (0, 3, 1, 2))
